# baseline (speedup 1.0000x reference)
import functools
import math

import jax
import jax.numpy as jnp
from jax import lax
from jax.experimental import pallas as pl
from jax.experimental.pallas import tpu as pltpu
from jax.experimental.pallas import tpu_sc as plsc

LOG_EPS = -64.0
EPS = float(math.exp(-64.0))
V, B, T, E, S = 1024, 32, 64, 4096, 4

_NC, _NS, _NL = 2, 16, 16
_E_PER_W = E // _NS
_CH = 128
_NCH = _E_PER_W // _CH
_ZB = 8192
_M_STRIPE = (V * V) // _NS
_OD_STRIPE = V // _NS
_NK = _M_STRIPE // _ZB


def _sc_build_body(in_hbm, out_hbm, zeros_hbm, ones_hbm, m_out, od_out,
                   m_sh, od_sh, obuf, ibuf, fbuf, ones, odbuf, stage, sem):
    cid = lax.axis_index("c")
    sid = lax.axis_index("s")

    wid = sid * _NC + cid

    def _build():
        pltpu.sync_copy(ones_hbm, ones)

        h = pltpu.async_copy(
            zeros_hbm, m_sh.at[pl.ds(sid * _M_STRIPE, _M_STRIPE)], sem)
        @pl.when(sid == 0)
        def _():
            pltpu.sync_copy(zeros_hbm.at[pl.ds(0, V)], od_sh)
        h.wait()
        plsc.subcore_barrier()

        for ch in range(_NCH):
            base = sid * _E_PER_W + ch * _CH
            pltpu.sync_copy(out_hbm.at[pl.ds(base, _CH)], obuf.at[ch])
            pltpu.sync_copy(in_hbm.at[pl.ds(base, _CH)], ibuf.at[ch])
            for g in range(_CH // _NL):
                o = obuf[ch, pl.ds(g * _NL, _NL)]
                i = ibuf[ch, pl.ds(g * _NL, _NL)]
                fbuf[ch, pl.ds(g * _NL, _NL)] = o * V + i
            pltpu.sync_copy(ones, m_sh.at[fbuf.at[ch]], add=True)
            pltpu.sync_copy(ones, od_sh.at[obuf.at[ch]], add=True)
        plsc.subcore_barrier()

        handles = []
        for k in range(_NK // 2):
            off = wid * (_M_STRIPE // 2) + k * _ZB
            slot = (k % 2) * _ZB
            if k >= 2:
                handles[k - 2].wait()
            pltpu.sync_copy(m_sh.at[pl.ds(off, _ZB)],
                            stage.at[pl.ds(slot, _ZB)])
            handles.append(
                pltpu.async_copy(stage.at[pl.ds(slot, _ZB)],
                                 m_out.at[pl.ds(off, _ZB)], sem))
        @pl.when(wid == 0)
        def _():
            pltpu.sync_copy(od_sh, odbuf)
            pltpu.sync_copy(odbuf, od_out)
        for h in handles[-2:]:
            h.wait()

    _build()


@functools.lru_cache(maxsize=1)
def _get_sc_build():
    return pl.kernel(
        _sc_build_body,
        out_type=(jax.ShapeDtypeStruct((V * V,), jnp.float32),
                  jax.ShapeDtypeStruct((V,), jnp.float32)),
        mesh=plsc.VectorSubcoreMesh(core_axis_name="c", subcore_axis_name="s"),
        scratch_types=[
            pltpu.VMEM_SHARED((V * V,), jnp.float32),
            pltpu.VMEM_SHARED((V,), jnp.float32),
            pltpu.VMEM((_NCH, _CH), jnp.int32),
            pltpu.VMEM((_NCH, _CH), jnp.int32),
            pltpu.VMEM((_NCH, _CH), jnp.int32),
            pltpu.VMEM((_CH,), jnp.float32),
            pltpu.VMEM((V,), jnp.float32),
            pltpu.VMEM((2 * _ZB,), jnp.float32),
            pltpu.SemaphoreType.DMA,
        ],
    )


def _tc_body(x_ref, m_ref, od_ref, lenc_ref, lenr_ref, s_ref, e_ref,
             out_ref, mb_ref):
    colv = lax.broadcasted_iota(jnp.int32, (1, V), 1)
    smask = (colv == s_ref[0]) | (colv == s_ref[1]) \
        | (colv == s_ref[2]) | (colv == s_ref[3])
    ecnt = ((colv == e_ref[0]).astype(jnp.float32)
            + (colv == e_ref[1]).astype(jnp.float32)
            + (colv == e_ref[2]).astype(jnp.float32)
            + (colv == e_ref[3]).astype(jnp.float32))
    lc = lenc_ref[...]
    lr = lenr_ref[...]
    bi = lax.broadcasted_iota(jnp.int32, (B, B), 0)
    bj = lax.broadcasted_iota(jnp.int32, (B, B), 1)
    eq = (lc == lr) & (bj <= bi)
    src = jnp.sum(eq.astype(jnp.int32), axis=1, keepdims=True) - 1
    sel = (bj == src).astype(jnp.float32)
    om = (od_ref[...] > 0).astype(jnp.float32)
    mb_ref[...] = m_ref[...].astype(jnp.bfloat16)

    def _select(s4, acc, res, t):
        log_last = jnp.log(s4) + acc
        gathered = jax.lax.dot_general(
            sel, log_last, (((1,), (0,)), ((), ())),
            precision=jax.lax.Precision.HIGHEST,
            preferred_element_type=jnp.float32)
        return jnp.where(lc == t + 1, gathered, res)

    pst = jnp.where(smask, jnp.exp(x_ref[0]), EPS)
    e = om * pst
    acc = jnp.zeros((B, 1), jnp.float32)
    res = _select(jnp.sum(pst * ecnt, axis=1, keepdims=True), acc,
                  jnp.zeros((B, 1), jnp.float32), 0)
    for t in range(1, T):
        s = jnp.sum(e, axis=1, keepdims=True)
        seps = s * EPS
        ef = jnp.maximum(e, seps).astype(jnp.bfloat16)
        a = jnp.dot(ef, mb_ref[...],
                    preferred_element_type=jnp.float32)
        rs_ex = jnp.exp(x_ref[t]) * (1.0 / s)
        g = om * rs_ex
        gt = ecnt * rs_ex
        acc = acc + jnp.log(s)
        ca = jnp.maximum(a, seps)
        e = g * ca
        res = _select(jnp.sum(gt * ca, axis=1, keepdims=True), acc, res, t)
    out_ref[...] = -res


def _tc_forward(xt, m_mat, odeg, lenc, lenr, start_idxs, end_idxs,
                interpret=False):
    return pl.pallas_call(
        _tc_body,
        in_specs=[
            pl.BlockSpec((T, B, V), lambda: (0, 0, 0)),
            pl.BlockSpec((V, V), lambda: (0, 0)),
            pl.BlockSpec((1, V), lambda: (0, 0)),
            pl.BlockSpec((B, 1), lambda: (0, 0)),
            pl.BlockSpec((1, B), lambda: (0, 0)),
            pl.BlockSpec(memory_space=pltpu.SMEM),
            pl.BlockSpec(memory_space=pltpu.SMEM),
        ],
        out_specs=pl.BlockSpec((B, 1), lambda: (0, 0)),
        out_shape=jax.ShapeDtypeStruct((B, 1), jnp.float32),
        scratch_shapes=[pltpu.VMEM((V, V), jnp.bfloat16)],
        interpret=interpret,
    )(xt, m_mat, odeg, lenc, lenr, start_idxs, end_idxs)


def kernel(extracted_log_probs, target_lengths, in_idxs, out_idxs,
           start_idxs, end_idxs):
    xt = jnp.transpose(extracted_log_probs, (2, 1, 0))
    m_flat, odeg = _get_sc_build()(
        in_idxs, out_idxs,
        jnp.zeros((_M_STRIPE,), jnp.float32),
        jnp.ones((_CH,), jnp.float32))
    out = _tc_forward(xt, m_flat.reshape(V, V), odeg.reshape(1, V),
                      target_lengths.reshape(B, 1),
                      target_lengths.reshape(1, B),
                      start_idxs, end_idxs)
    return out.reshape(B)


# speedup vs baseline: 1.0070x; 1.0070x over previous
import functools
import math

import jax
import jax.numpy as jnp
from jax import lax
from jax.experimental import pallas as pl
from jax.experimental.pallas import tpu as pltpu
from jax.experimental.pallas import tpu_sc as plsc

LOG_EPS = -64.0
EPS = float(math.exp(-64.0))
V, B, T, E, S = 1024, 32, 64, 4096, 4

_NC, _NS, _NL = 2, 16, 16
_E_PER_W = E // _NS
_CH = 128
_NCH = _E_PER_W // _CH
_ZB = 8192
_M_STRIPE = (V * V) // _NS
_OD_STRIPE = V // _NS
_NK = _M_STRIPE // _ZB


def _sc_build_body(in_hbm, out_hbm, zeros_hbm, ones_hbm, m_out, od_out,
                   m_sh, od_sh, obuf, ibuf, fbuf, ones, odbuf, stage, sem):
    cid = lax.axis_index("c")
    sid = lax.axis_index("s")

    wid = sid * _NC + cid

    def _build():
        pltpu.sync_copy(ones_hbm, ones)

        h = pltpu.async_copy(
            zeros_hbm, m_sh.at[pl.ds(sid * _M_STRIPE, _M_STRIPE)], sem)
        @pl.when(sid == 0)
        def _():
            pltpu.sync_copy(zeros_hbm.at[pl.ds(0, V)], od_sh)
        h.wait()
        plsc.subcore_barrier()

        for ch in range(_NCH):
            base = sid * _E_PER_W + ch * _CH
            pltpu.sync_copy(out_hbm.at[pl.ds(base, _CH)], obuf.at[ch])
            pltpu.sync_copy(in_hbm.at[pl.ds(base, _CH)], ibuf.at[ch])
            for g in range(_CH // _NL):
                o = obuf[ch, pl.ds(g * _NL, _NL)]
                i = ibuf[ch, pl.ds(g * _NL, _NL)]
                fbuf[ch, pl.ds(g * _NL, _NL)] = o * V + i
            pltpu.sync_copy(ones, m_sh.at[fbuf.at[ch]], add=True)
            pltpu.sync_copy(ones, od_sh.at[obuf.at[ch]], add=True)
        plsc.subcore_barrier()

        handles = []
        for k in range(_NK // 2):
            off = wid * (_M_STRIPE // 2) + k * _ZB
            slot = (k % 2) * _ZB
            if k >= 2:
                handles[k - 2].wait()
            pltpu.sync_copy(m_sh.at[pl.ds(off, _ZB)],
                            stage.at[pl.ds(slot, _ZB)])
            handles.append(
                pltpu.async_copy(stage.at[pl.ds(slot, _ZB)],
                                 m_out.at[pl.ds(off, _ZB)], sem))
        @pl.when(wid == 0)
        def _():
            pltpu.sync_copy(od_sh, odbuf)
            pltpu.sync_copy(odbuf, od_out)
        for h in handles[-2:]:
            h.wait()

    _build()


@functools.lru_cache(maxsize=1)
def _get_sc_build():
    return pl.kernel(
        _sc_build_body,
        out_type=(jax.ShapeDtypeStruct((V * V,), jnp.float32),
                  jax.ShapeDtypeStruct((V,), jnp.float32)),
        mesh=plsc.VectorSubcoreMesh(core_axis_name="c", subcore_axis_name="s"),
        scratch_types=[
            pltpu.VMEM_SHARED((V * V,), jnp.float32),
            pltpu.VMEM_SHARED((V,), jnp.float32),
            pltpu.VMEM((_NCH, _CH), jnp.int32),
            pltpu.VMEM((_NCH, _CH), jnp.int32),
            pltpu.VMEM((_NCH, _CH), jnp.int32),
            pltpu.VMEM((_CH,), jnp.float32),
            pltpu.VMEM((V,), jnp.float32),
            pltpu.VMEM((2 * _ZB,), jnp.float32),
            pltpu.SemaphoreType.DMA,
        ],
    )


def _tc_body(x_ref, m_ref, od_ref, lenc_ref, lenr_ref, s_ref, e_ref,
             out_ref):
    colv = lax.broadcasted_iota(jnp.int32, (1, V), 1)
    smask = (colv == s_ref[0]) | (colv == s_ref[1]) \
        | (colv == s_ref[2]) | (colv == s_ref[3])
    ecnt = ((colv == e_ref[0]).astype(jnp.float32)
            + (colv == e_ref[1]).astype(jnp.float32)
            + (colv == e_ref[2]).astype(jnp.float32)
            + (colv == e_ref[3]).astype(jnp.float32))
    lc = lenc_ref[...]
    lr = lenr_ref[...]
    bi = lax.broadcasted_iota(jnp.int32, (B, B), 0)
    bj = lax.broadcasted_iota(jnp.int32, (B, B), 1)
    eq = (lc == lr) & (bj <= bi)
    src = jnp.sum(eq.astype(jnp.int32), axis=1, keepdims=True) - 1
    sel = (bj == src).astype(jnp.float32)
    om = (od_ref[...] > 0).astype(jnp.float32)

    def _select(s4, acc, res, t):
        log_last = jnp.log(s4) + acc
        gathered = jax.lax.dot_general(
            sel, log_last, (((1,), (0,)), ((), ())),
            precision=jax.lax.Precision.HIGHEST,
            preferred_element_type=jnp.float32)
        return jnp.where(lc == t + 1, gathered, res)

    def _lanesum(v):
        y = v[:, 0:128]
        for k in range(1, V // 128):
            y = y + v[:, 128 * k:128 * (k + 1)]
        for k in (64, 32, 16, 8, 4, 2, 1):
            y = y + pltpu.roll(y, k, axis=1)
        return y[:, 0:1]

    pst = jnp.where(smask, jnp.exp(x_ref[0]), EPS)
    e = om * pst
    acc = jnp.zeros((B, 1), jnp.float32)
    res = _select(jnp.sum(pst * ecnt, axis=1, keepdims=True), acc,
                  jnp.zeros((B, 1), jnp.float32), 0)
    for t in range(1, T):
        s = jnp.sum(e, axis=1, keepdims=True)
        seps = s * EPS
        ef = jnp.maximum(e, seps).astype(jnp.bfloat16)
        a = jnp.dot(ef, m_ref[...],
                    preferred_element_type=jnp.float32)
        rs_ex = jnp.exp(x_ref[t]) * (1.0 / s)
        g = om * rs_ex
        gt = ecnt * rs_ex
        acc = acc + jnp.log(s)
        ca = jnp.maximum(a, seps)
        e = g * ca
        res = _select(jnp.sum(gt * ca, axis=1, keepdims=True), acc, res, t)
    out_ref[...] = -res


def _tc_forward(xt, m_mat, odeg, lenc, lenr, start_idxs, end_idxs,
                interpret=False):
    return pl.pallas_call(
        _tc_body,
        in_specs=[
            pl.BlockSpec((T, B, V), lambda: (0, 0, 0)),
            pl.BlockSpec((V, V), lambda: (0, 0)),
            pl.BlockSpec((1, V), lambda: (0, 0)),
            pl.BlockSpec((B, 1), lambda: (0, 0)),
            pl.BlockSpec((1, B), lambda: (0, 0)),
            pl.BlockSpec(memory_space=pltpu.SMEM),
            pl.BlockSpec(memory_space=pltpu.SMEM),
        ],
        out_specs=pl.BlockSpec((B, 1), lambda: (0, 0)),
        out_shape=jax.ShapeDtypeStruct((B, 1), jnp.float32),
        interpret=interpret,
    )(xt, m_mat, odeg, lenc, lenr, start_idxs, end_idxs)


def kernel(extracted_log_probs, target_lengths, in_idxs, out_idxs,
           start_idxs, end_idxs):
    xt = jnp.transpose(extracted_log_probs, (2, 1, 0))
    m_flat, odeg = _get_sc_build()(
        in_idxs, out_idxs,
        jnp.zeros((_M_STRIPE,), jnp.float32),
        jnp.ones((_CH,), jnp.float32))
    out = _tc_forward(xt, m_flat.reshape(V, V).astype(jnp.bfloat16),
                      odeg.reshape(1, V),
                      target_lengths.reshape(B, 1),
                      target_lengths.reshape(1, B),
                      start_idxs, end_idxs)
    return out.reshape(B)


# speedup vs baseline: 1.0140x; 1.0070x over previous
import functools
import math

import jax
import jax.numpy as jnp
from jax import lax
from jax.experimental import pallas as pl
from jax.experimental.pallas import tpu as pltpu
from jax.experimental.pallas import tpu_sc as plsc

LOG_EPS = -64.0
EPS = float(math.exp(-64.0))
V, B, T, E, S = 1024, 32, 64, 4096, 4

_NC, _NS, _NL = 2, 16, 16
_E_PER_W = E // _NS
_CH = 128
_NCH = _E_PER_W // _CH
_ZB = 8192
_M_STRIPE = (V * V) // _NS
_OD_STRIPE = V // _NS
_NK = _M_STRIPE // _ZB


def _sc_build_body(in_hbm, out_hbm, zeros_hbm, ones_hbm, m_out, od_out,
                   m_sh, od_sh, obuf, ibuf, fbuf, ones, odbuf, stage, sem):
    cid = lax.axis_index("c")
    sid = lax.axis_index("s")

    @pl.when(cid == 0)
    def _build():
        pltpu.sync_copy(ones_hbm, ones)

        h = pltpu.async_copy(
            zeros_hbm, m_sh.at[pl.ds(sid * _M_STRIPE, _M_STRIPE)], sem)
        @pl.when(sid == 0)
        def _():
            pltpu.sync_copy(zeros_hbm.at[pl.ds(0, V)], od_sh)
        h.wait()
        plsc.subcore_barrier()

        for ch in range(_NCH):
            base = sid * _E_PER_W + ch * _CH
            pltpu.sync_copy(out_hbm.at[pl.ds(base, _CH)], obuf.at[ch])
            pltpu.sync_copy(in_hbm.at[pl.ds(base, _CH)], ibuf.at[ch])
            for g in range(_CH // _NL):
                o = obuf[ch, pl.ds(g * _NL, _NL)]
                i = ibuf[ch, pl.ds(g * _NL, _NL)]
                fbuf[ch, pl.ds(g * _NL, _NL)] = o * V + i
            pltpu.sync_copy(ones, m_sh.at[fbuf.at[ch]], add=True)
            pltpu.sync_copy(ones, od_sh.at[obuf.at[ch]], add=True)
        plsc.subcore_barrier()

        handles = []
        for k in range(_NK):
            off = sid * _M_STRIPE + k * _ZB
            slot = (k % 2) * _ZB
            if k >= 2:
                handles[k - 2].wait()
            pltpu.sync_copy(m_sh.at[pl.ds(off, _ZB)],
                            stage.at[pl.ds(slot, _ZB)])
            handles.append(
                pltpu.async_copy(stage.at[pl.ds(slot, _ZB)],
                                 m_out.at[pl.ds(off, _ZB)], sem))
        @pl.when(sid == 0)
        def _():
            pltpu.sync_copy(od_sh, odbuf)
            pltpu.sync_copy(odbuf, od_out)
        for h in handles[-2:]:
            h.wait()


@functools.lru_cache(maxsize=1)
def _get_sc_build():
    return pl.kernel(
        _sc_build_body,
        out_type=(jax.ShapeDtypeStruct((V * V,), jnp.float32),
                  jax.ShapeDtypeStruct((V,), jnp.float32)),
        mesh=plsc.VectorSubcoreMesh(core_axis_name="c", subcore_axis_name="s"),
        scratch_types=[
            pltpu.VMEM_SHARED((V * V,), jnp.float32),
            pltpu.VMEM_SHARED((V,), jnp.float32),
            pltpu.VMEM((_NCH, _CH), jnp.int32),
            pltpu.VMEM((_NCH, _CH), jnp.int32),
            pltpu.VMEM((_NCH, _CH), jnp.int32),
            pltpu.VMEM((_CH,), jnp.float32),
            pltpu.VMEM((V,), jnp.float32),
            pltpu.VMEM((2 * _ZB,), jnp.float32),
            pltpu.SemaphoreType.DMA,
        ],
    )


def _tc_body(x_ref, m_ref, od_ref, lenc_ref, lenr_ref, s_ref, e_ref,
             out_ref):
    colv = lax.broadcasted_iota(jnp.int32, (1, V), 1)
    smask = (colv == s_ref[0]) | (colv == s_ref[1]) \
        | (colv == s_ref[2]) | (colv == s_ref[3])
    ecnt = ((colv == e_ref[0]).astype(jnp.float32)
            + (colv == e_ref[1]).astype(jnp.float32)
            + (colv == e_ref[2]).astype(jnp.float32)
            + (colv == e_ref[3]).astype(jnp.float32))
    lc = lenc_ref[...]
    lr = lenr_ref[...]
    bi = lax.broadcasted_iota(jnp.int32, (B, B), 0)
    bj = lax.broadcasted_iota(jnp.int32, (B, B), 1)
    eq = (lc == lr) & (bj <= bi)
    src = jnp.sum(eq.astype(jnp.int32), axis=1, keepdims=True) - 1
    sel = (bj == src).astype(jnp.float32)
    om = (od_ref[...] > 0).astype(jnp.float32)

    def _select(s4, acc, res, t):
        log_last = jnp.log(s4) + acc
        gathered = jax.lax.dot_general(
            sel, log_last, (((1,), (0,)), ((), ())),
            precision=jax.lax.Precision.HIGHEST,
            preferred_element_type=jnp.float32)
        return jnp.where(lc == t + 1, gathered, res)

    def _lanesum(v):
        y = v[:, 0:128]
        for k in range(1, V // 128):
            y = y + v[:, 128 * k:128 * (k + 1)]
        for k in (64, 32, 16, 8, 4, 2, 1):
            y = y + pltpu.roll(y, k, axis=1)
        return y[:, 0:1]

    pst = jnp.where(smask, jnp.exp(x_ref[0]), EPS)
    e = om * pst
    acc = jnp.zeros((B, 1), jnp.float32)
    res = _select(jnp.sum(pst * ecnt, axis=1, keepdims=True), acc,
                  jnp.zeros((B, 1), jnp.float32), 0)
    for t in range(1, T):
        s = jnp.sum(e, axis=1, keepdims=True)
        seps = s * EPS
        ef = jnp.maximum(e, seps).astype(jnp.bfloat16)
        a = jnp.dot(ef, m_ref[...],
                    preferred_element_type=jnp.float32)
        rs_ex = jnp.exp(x_ref[t]) * (1.0 / s)
        g = om * rs_ex
        gt = ecnt * rs_ex
        acc = acc + jnp.log(s)
        ca = jnp.maximum(a, seps)
        e = g * ca
        res = _select(jnp.sum(gt * ca, axis=1, keepdims=True), acc, res, t)
    out_ref[...] = -res


def _tc_forward(xt, m_mat, odeg, lenc, lenr, start_idxs, end_idxs,
                interpret=False):
    return pl.pallas_call(
        _tc_body,
        in_specs=[
            pl.BlockSpec((T, B, V), lambda: (0, 0, 0)),
            pl.BlockSpec((V, V), lambda: (0, 0)),
            pl.BlockSpec((1, V), lambda: (0, 0)),
            pl.BlockSpec((B, 1), lambda: (0, 0)),
            pl.BlockSpec((1, B), lambda: (0, 0)),
            pl.BlockSpec(memory_space=pltpu.SMEM),
            pl.BlockSpec(memory_space=pltpu.SMEM),
        ],
        out_specs=pl.BlockSpec((B, 1), lambda: (0, 0)),
        out_shape=jax.ShapeDtypeStruct((B, 1), jnp.float32),
        interpret=interpret,
    )(xt, m_mat, odeg, lenc, lenr, start_idxs, end_idxs)


def kernel(extracted_log_probs, target_lengths, in_idxs, out_idxs,
           start_idxs, end_idxs):
    xt = jnp.transpose(extracted_log_probs, (2, 1, 0))
    m_flat, odeg = _get_sc_build()(
        in_idxs, out_idxs,
        jnp.zeros((_M_STRIPE,), jnp.float32),
        jnp.ones((_CH,), jnp.float32))
    out = _tc_forward(xt, m_flat.reshape(V, V).astype(jnp.bfloat16),
                      odeg.reshape(1, V),
                      target_lengths.reshape(B, 1),
                      target_lengths.reshape(1, B),
                      start_idxs, end_idxs)
    return out.reshape(B)
